# Initial kernel scaffold; baseline (speedup 1.0000x reference)
import jax
import jax.numpy as jnp
from jax.experimental import pallas as pl


def kernel(query_embedding, gate_W, gate_b, W1, b1, W2, b2):
    raise NotImplementedError("write your pallas kernel here")


# speedup vs baseline: 9.9836x; 9.9836x over previous
import functools

import jax
import jax.numpy as jnp
from jax.experimental import pallas as pl
from jax.experimental.pallas import tpu as pltpu

_H_PHASOR = 192
_DELTA0 = 7.0
_TOPK_FEAT = 32
_DT = 0.001
_TAU = 0.02


def _gate_logits(x, gate_W, gate_b):
    q = jnp.mean(x, axis=-1)
    freqs = _DELTA0 * jnp.arange(1, _H_PHASOR + 1, dtype=jnp.float32)
    ang = q[:, None] * freqs[None, :]
    tf = jnp.concatenate([jnp.cos(ang), jnp.sin(ang)], axis=-1)
    s0 = jnp.mean(tf, axis=-1)
    w = jnp.exp(-jnp.arange(_TOPK_FEAT, dtype=jnp.float32) * _DT / _TAU)
    s1 = jnp.full_like(q, jnp.sum(w) / x.shape[-1])
    gate_inputs = jnp.stack([s0, s1], axis=-1)
    return gate_inputs @ gate_W + gate_b


def _moe_kernel(gl_ref, x_ref, w1_ref, b1_ref, w2_ref, b2_ref, out_ref):
    tb, d = x_ref.shape
    e_num = gl_ref.shape[1]

    gl = gl_ref[...]
    m = jnp.max(gl, axis=1, keepdims=True)
    eg = jnp.exp(gl - m)
    p = eg / jnp.sum(eg, axis=1, keepdims=True)

    col = jax.lax.broadcasted_iota(jnp.int32, (tb, e_num), 1)
    i1 = jnp.min(jnp.where(gl == m, col, e_num), axis=1, keepdims=True)
    mask1 = col == i1
    gl2 = jnp.where(mask1, -jnp.inf, gl)
    m2 = jnp.max(gl2, axis=1, keepdims=True)
    i2 = jnp.min(jnp.where(gl2 == m2, col, e_num), axis=1, keepdims=True)
    mask = mask1 | (col == i2)

    gated = jnp.where(mask, p, 0.0)
    g = gated / (jnp.sum(gated, axis=1, keepdims=True) + 1e-9)

    xb = x_ref[...]
    acc = jnp.zeros((tb, d), jnp.float32)
    for e in range(e_num):
        h = jnp.dot(xb, w1_ref[e], preferred_element_type=jnp.float32)
        h = jax.nn.gelu(h + b1_ref[e][None, :])
        o = jnp.dot(h.astype(jnp.bfloat16), w2_ref[e],
                    preferred_element_type=jnp.float32)
        o = o + b2_ref[e][None, :]
        acc = acc + g[:, e:e + 1] * o
    out_ref[...] = acc


@jax.jit
def kernel(query_embedding, gate_W, gate_b, W1, b1, W2, b2):
    x = query_embedding
    b_sz, d = x.shape
    e_num, _, f = W1.shape

    gl = _gate_logits(x, gate_W, gate_b)
    x16 = x.astype(jnp.bfloat16)
    w1 = W1.astype(jnp.bfloat16)
    w2 = W2.astype(jnp.bfloat16)

    tb = 256 if b_sz % 256 == 0 else b_sz
    grid = (b_sz // tb,)

    out = pl.pallas_call(
        _moe_kernel,
        grid=grid,
        in_specs=[
            pl.BlockSpec((tb, e_num), lambda i: (i, 0)),
            pl.BlockSpec((tb, d), lambda i: (i, 0)),
            pl.BlockSpec((e_num, d, f), lambda i: (0, 0, 0)),
            pl.BlockSpec((e_num, f), lambda i: (0, 0)),
            pl.BlockSpec((e_num, f, d), lambda i: (0, 0, 0)),
            pl.BlockSpec((e_num, d), lambda i: (0, 0)),
        ],
        out_specs=pl.BlockSpec((tb, d), lambda i: (i, 0)),
        out_shape=jax.ShapeDtypeStruct((b_sz, d), jnp.float32),
        compiler_params=pltpu.CompilerParams(
            dimension_semantics=("arbitrary",),
        ),
    )(gl, x16, w1, b1, w2, b2)
    return out
